# baseline (speedup 1.0000x reference)
import dataclasses
import functools

import numpy as np
import jax
import jax.numpy as jnp
from jax import lax
from jax.experimental import pallas as pl
from jax.experimental.pallas import tpu as pltpu
from jax.experimental.pallas import tpu_sc as plsc

VOCAB = 100000
D_MODEL = 1024
MAX_LEN = 2048
PAD_IDX = 0

NC = 2
NS = 16
NW = NC * NS
CHUNK = 16
NBUF = 4
K_SLICES = 2
TC_BLK = 512


def _pos_encoding(max_len, d_model):
    enc = np.zeros((max_len, d_model), dtype=np.float32)
    pos = np.arange(0, max_len, dtype=np.float32)[:, None]
    _2i = np.arange(0, d_model, 2, dtype=np.float32)
    enc[:, 0::2] = np.sin(pos / 10000 ** (_2i / d_model))
    enc[:, 1::2] = np.cos(pos / 10000 ** (_2i / d_model))
    return jnp.asarray(enc)


_POS_ENC = _pos_encoding(MAX_LEN, D_MODEL)


def _sc_gather(x_slice, table, d_model):
    n_tok = x_slice.shape[0]
    b_per_w = n_tok // NW
    n_chunks = b_per_w // CHUNK

    mesh = plsc.VectorSubcoreMesh(core_axis_name="c", subcore_axis_name="s")
    cp = pltpu.CompilerParams()
    if "needs_layout_passes" in pltpu.CompilerParams.__dataclass_fields__:
        cp = dataclasses.replace(cp, needs_layout_passes=False)

    @functools.partial(
        pl.kernel,
        out_type=jax.ShapeDtypeStruct((n_tok, d_model), jnp.float32),
        mesh=mesh,
        compiler_params=cp,
        scratch_types=[
            pltpu.VMEM((b_per_w,), jnp.int32),
            pltpu.VMEM((NBUF, CHUNK, d_model), jnp.float32),
            pltpu.SemaphoreType.DMA((NBUF,)),
            pltpu.SemaphoreType.DMA((NBUF,)),
        ],
    )
    def gather_k(table_hbm, xf_hbm, o_hbm, idx_v, gbuf, sem_g, sem_o):
        wid = lax.axis_index("s") * NC + lax.axis_index("c")
        base = wid * b_per_w
        pltpu.sync_copy(xf_hbm.at[pl.ds(base, b_per_w)], idx_v)

        def gather_copy(c, b):
            return pltpu.make_async_copy(
                table_hbm.at[idx_v.at[pl.ds(c * CHUNK, CHUNK)]],
                gbuf.at[b], sem_g.at[b],
            )

        def out_copy(c, b):
            return pltpu.make_async_copy(
                gbuf.at[b], o_hbm.at[pl.ds(base + c * CHUNK, CHUNK)],
                sem_o.at[b],
            )

        for b in range(NBUF):
            gather_copy(b, b).start()

        @pl.loop(0, n_chunks, step=NBUF)
        def _step(i):
            for b in range(NBUF):
                c = i + b
                gather_copy(c, b).wait()
                out_copy(c, b).start()

                @pl.when(c + NBUF < n_chunks)
                def _():
                    out_copy(c, b).wait()
                    gather_copy(c + NBUF, b).start()

        for b in range(NBUF):
            out_copy(n_chunks - NBUF + b, b).wait()

    return gather_k(table, x_slice)


def _tc_mask_add(t, x_slice, pos_slice, d_model):
    n_tok = t.shape[0]
    n_blk = n_tok // TC_BLK
    x3 = x_slice.reshape(n_blk, TC_BLK, 1)
    pos_blocks = pos_slice.shape[0] // TC_BLK

    def body(t_ref, x_ref, p_ref, o_ref):
        keep = x_ref[0] != PAD_IDX
        o_ref[...] = jnp.where(keep, t_ref[...], 0.0) + p_ref[...]

    return pl.pallas_call(
        body,
        out_shape=jax.ShapeDtypeStruct((n_tok, d_model), jnp.float32),
        grid=(n_blk,),
        in_specs=[
            pl.BlockSpec((TC_BLK, d_model), lambda i: (i, 0)),
            pl.BlockSpec((1, TC_BLK, 1), lambda i: (i, 0, 0)),
            pl.BlockSpec((TC_BLK, d_model), lambda i: (i % pos_blocks, 0)),
        ],
        out_specs=pl.BlockSpec((TC_BLK, d_model), lambda i: (i, 0)),
    )(t, x3, pos_slice)


def kernel(x, table):
    batch, seq_len = x.shape
    d_model = table.shape[1]
    n_tok = batch * seq_len
    pos = _POS_ENC[:seq_len, :]
    x_flat = x.reshape(-1)

    per_slice = n_tok // K_SLICES
    outs = []
    for k in range(K_SLICES):
        x_k = lax.slice(x_flat, (k * per_slice,), ((k + 1) * per_slice,))
        t_k = _sc_gather(x_k, table, d_model)
        outs.append(_tc_mask_add(t_k, x_k, pos, d_model))

    out = jnp.concatenate(outs, axis=0)
    return out.reshape(batch, seq_len, d_model)


# speedup vs baseline: 1.3612x; 1.3612x over previous
import dataclasses
import functools

import numpy as np
import jax
import jax.numpy as jnp
from jax import lax
from jax.experimental import pallas as pl
from jax.experimental.pallas import tpu as pltpu
from jax.experimental.pallas import tpu_sc as plsc

VOCAB = 100000
D_MODEL = 1024
MAX_LEN = 2048
PAD_IDX = 0

LANES = 16
NC = 2
NS = 16
NW = NC * NS
CHUNK = 16
NBUF = 2


def _pos_encoding(max_len, d_model):
    enc = np.zeros((max_len, d_model), dtype=np.float32)
    pos = np.arange(0, max_len, dtype=np.float32)[:, None]
    _2i = np.arange(0, d_model, 2, dtype=np.float32)
    enc[:, 0::2] = np.sin(pos / 10000 ** (_2i / d_model))
    enc[:, 1::2] = np.cos(pos / 10000 ** (_2i / d_model))
    return jnp.asarray(enc)


_POS_ENC = _pos_encoding(MAX_LEN, D_MODEL)


def kernel(x, table):
    batch, seq_len = x.shape
    d_model = table.shape[1]
    n_tok = batch * seq_len
    b_per_w = n_tok // NW
    n_chunks = b_per_w // CHUNK
    pos = _POS_ENC[:seq_len, :]
    x_flat = x.reshape(-1)

    mesh = plsc.VectorSubcoreMesh(core_axis_name="c", subcore_axis_name="s")
    cp = pltpu.CompilerParams()
    if "needs_layout_passes" in pltpu.CompilerParams.__dataclass_fields__:
        cp = dataclasses.replace(cp, needs_layout_passes=False)

    @functools.partial(
        pl.kernel,
        out_type=jax.ShapeDtypeStruct((n_tok, d_model), jnp.float32),
        mesh=mesh,
        compiler_params=cp,
        scratch_types=[
            pltpu.VMEM((b_per_w,), jnp.int32),
            pltpu.VMEM((NBUF, CHUNK, d_model), jnp.float32),
            pltpu.VMEM((NBUF, CHUNK, d_model), jnp.float32),
            pltpu.SemaphoreType.DMA((NBUF,)),
            pltpu.SemaphoreType.DMA((NBUF,)),
            pltpu.SemaphoreType.DMA((NBUF,)),
        ],
    )
    def embed(table_hbm, xf_hbm, pos_hbm, o_hbm,
              idx_v, gbuf, obuf, sem_g, sem_p, sem_o):
        wid = lax.axis_index("s") * NC + lax.axis_index("c")
        base = wid * b_per_w
        pos_base = lax.rem(base, seq_len)
        pltpu.sync_copy(xf_hbm.at[pl.ds(base, b_per_w)], idx_v)

        def gather_copy(c, b):
            return pltpu.make_async_copy(
                table_hbm.at[idx_v.at[pl.ds(c * CHUNK, CHUNK)]],
                gbuf.at[b], sem_g.at[b],
            )

        def pos_copy(c, b):
            return pltpu.make_async_copy(
                pos_hbm.at[pl.ds(pos_base + c * CHUNK, CHUNK)],
                obuf.at[b], sem_p.at[b],
            )

        def out_copy(c, b):
            return pltpu.make_async_copy(
                obuf.at[b], o_hbm.at[pl.ds(base + c * CHUNK, CHUNK)],
                sem_o.at[b],
            )

        def compute_chunk(c, b):
            row0 = c * CHUNK
            idx_chunk = idx_v.at[pl.ds(row0, CHUNK)][...]
            has_pad = jnp.any(idx_chunk == PAD_IDX)

            @pl.when(jnp.logical_not(has_pad))
            def _fast():
                @pl.loop(0, CHUNK)
                def _row(r):
                    for c0 in range(0, d_model, LANES):
                        sl = pl.ds(c0, LANES)
                        plsc.addupdate(
                            obuf.at[b, r, sl], gbuf.at[b, r, sl][...]
                        )

            @pl.when(has_pad)
            def _masked():
                @pl.loop(0, CHUNK)
                def _row(r):
                    idx_r = plsc.load_gather(
                        idx_v, [jnp.full((LANES,), row0 + r, jnp.int32)]
                    )
                    scale = jnp.where(
                        idx_r != PAD_IDX,
                        jnp.ones((LANES,), jnp.float32),
                        jnp.zeros((LANES,), jnp.float32),
                    )
                    for c0 in range(0, d_model, LANES):
                        sl = pl.ds(c0, LANES)
                        plsc.addupdate(
                            obuf.at[b, r, sl],
                            gbuf.at[b, r, sl][...] * scale,
                        )

        for b in range(NBUF):
            gather_copy(b, b).start()
            pos_copy(b, b).start()

        @pl.loop(0, n_chunks, step=NBUF)
        def _step(i):
            for b in range(NBUF):
                c = i + b
                gather_copy(c, b).wait()
                pos_copy(c, b).wait()
                compute_chunk(c, b)
                out_copy(c, b).start()

                @pl.when(c + NBUF < n_chunks)
                def _():
                    out_copy(c, b).wait()
                    gather_copy(c + NBUF, b).start()
                    pos_copy(c + NBUF, b).start()

        for b in range(NBUF):
            out_copy(n_chunks - NBUF + b, b).wait()

    out = embed(table, x_flat, pos)
    return out.reshape(batch, seq_len, d_model)


# speedup vs baseline: 1.5326x; 1.1259x over previous
import dataclasses
import functools

import numpy as np
import jax
import jax.numpy as jnp
from jax import lax
from jax.experimental import pallas as pl
from jax.experimental.pallas import tpu as pltpu
from jax.experimental.pallas import tpu_sc as plsc

VOCAB = 100000
D_MODEL = 1024
MAX_LEN = 2048
PAD_IDX = 0

LANES = 16
NC = 2
NS = 16
NW = NC * NS
CHUNK = 8
NBUF = 4


def _pos_encoding(max_len, d_model):
    enc = np.zeros((max_len, d_model), dtype=np.float32)
    pos = np.arange(0, max_len, dtype=np.float32)[:, None]
    _2i = np.arange(0, d_model, 2, dtype=np.float32)
    enc[:, 0::2] = np.sin(pos / 10000 ** (_2i / d_model))
    enc[:, 1::2] = np.cos(pos / 10000 ** (_2i / d_model))
    return jnp.asarray(enc)


_POS_ENC = _pos_encoding(MAX_LEN, D_MODEL)


def kernel(x, table):
    batch, seq_len = x.shape
    d_model = table.shape[1]
    n_tok = batch * seq_len
    b_per_w = n_tok // NW
    n_chunks = b_per_w // CHUNK
    pos = _POS_ENC[:seq_len, :]
    x_flat = x.reshape(-1)

    mesh = plsc.VectorSubcoreMesh(core_axis_name="c", subcore_axis_name="s")
    cp = pltpu.CompilerParams()
    if "needs_layout_passes" in pltpu.CompilerParams.__dataclass_fields__:
        cp = dataclasses.replace(cp, needs_layout_passes=False)

    @functools.partial(
        pl.kernel,
        out_type=jax.ShapeDtypeStruct((n_tok, d_model), jnp.float32),
        mesh=mesh,
        compiler_params=cp,
        scratch_types=[
            pltpu.VMEM((b_per_w,), jnp.int32),
            pltpu.VMEM((NBUF, CHUNK, d_model), jnp.float32),
            pltpu.VMEM((NBUF, CHUNK, d_model), jnp.float32),
            pltpu.SemaphoreType.DMA((NBUF,)),
            pltpu.SemaphoreType.DMA((NBUF,)),
            pltpu.SemaphoreType.DMA((NBUF,)),
        ],
    )
    def embed(table_hbm, xf_hbm, pos_hbm, o_hbm,
              idx_v, gbuf, obuf, sem_g, sem_p, sem_o):
        wid = lax.axis_index("s") * NC + lax.axis_index("c")
        base = wid * b_per_w
        pos_base = lax.rem(base, seq_len)
        pltpu.sync_copy(xf_hbm.at[pl.ds(base, b_per_w)], idx_v)

        def gather_copy(c, b):
            return pltpu.make_async_copy(
                table_hbm.at[idx_v.at[pl.ds(c * CHUNK, CHUNK)]],
                gbuf.at[b], sem_g.at[b],
            )

        def pos_copy(c, b):
            return pltpu.make_async_copy(
                pos_hbm.at[pl.ds(pos_base + c * CHUNK, CHUNK)],
                obuf.at[b], sem_p.at[b],
            )

        def out_copy(c, b):
            return pltpu.make_async_copy(
                obuf.at[b], o_hbm.at[pl.ds(base + c * CHUNK, CHUNK)],
                sem_o.at[b],
            )

        def compute_chunk(c, b):
            row0 = c * CHUNK
            win0 = (row0 // LANES) * LANES
            idx_chunk = idx_v.at[pl.ds(win0, LANES)][...]
            has_pad = jnp.any(idx_chunk == PAD_IDX)

            @pl.when(jnp.logical_not(has_pad))
            def _fast():
                @pl.loop(0, CHUNK)
                def _row(r):
                    for c0 in range(0, d_model, LANES):
                        sl = pl.ds(c0, LANES)
                        plsc.addupdate(
                            obuf.at[b, r, sl], gbuf.at[b, r, sl][...]
                        )

            @pl.when(has_pad)
            def _masked():
                @pl.loop(0, CHUNK)
                def _row(r):
                    idx_r = plsc.load_gather(
                        idx_v, [jnp.full((LANES,), row0 + r, jnp.int32)]
                    )
                    scale = jnp.where(
                        idx_r != PAD_IDX,
                        jnp.ones((LANES,), jnp.float32),
                        jnp.zeros((LANES,), jnp.float32),
                    )
                    for c0 in range(0, d_model, LANES):
                        sl = pl.ds(c0, LANES)
                        plsc.addupdate(
                            obuf.at[b, r, sl],
                            gbuf.at[b, r, sl][...] * scale,
                        )

        for b in range(NBUF):
            gather_copy(b, b).start()
            pos_copy(b, b).start()

        @pl.loop(0, n_chunks, step=NBUF)
        def _step(i):
            for b in range(NBUF):
                c = i + b
                gather_copy(c, b).wait()
                pos_copy(c, b).wait()
                compute_chunk(c, b)
                out_copy(c, b).start()

                @pl.when(c + NBUF < n_chunks)
                def _():
                    out_copy(c, b).wait()
                    gather_copy(c + NBUF, b).start()
                    pos_copy(c + NBUF, b).start()

        for b in range(NBUF):
            out_copy(n_chunks - NBUF + b, b).wait()

    out = embed(table, x_flat, pos)
    return out.reshape(batch, seq_len, d_model)
